# baseline (speedup 1.0000x reference)
import functools
import jax
import jax.numpy as jnp
from jax import lax
from jax.experimental import pallas as pl
from jax.experimental.pallas import tpu as pltpu
from jax.experimental.pallas import tpu_sc as plsc

NE = 8
NR = 7
CAP = 256
SEQ = 2048
HID = 1024
INTER = 2048
TILE = 256

NC = 2
NS = 16
NW = NC * NS
GROWS = NR * CAP
TPW = SEQ // NW


def _router_body(x_ref, wr_ref, br_ref, gidx_ref, sw_ref, noopw_ref,
                 rv0_ref, rv1_ref, rm0_ref, rm1_ref):
    x = x_ref[...]
    logits = lax.dot_general(
        x, wr_ref[...], (((1,), (1,)), ((), ())),
        preferred_element_type=jnp.float32) + br_ref[...]
    m = jnp.max(logits, axis=1, keepdims=True)
    ex = jnp.exp(logits - m)
    p = ex / jnp.sum(ex, axis=1, keepdims=True)

    colid = lax.broadcasted_iota(jnp.int32, (SEQ, NE), 1)
    m1 = jnp.max(p, axis=1, keepdims=True)
    j1 = jnp.min(jnp.where(p == m1, colid, NE), axis=1, keepdims=True)
    sel1 = colid == j1
    p2 = jnp.where(sel1, -jnp.inf, p)
    m2 = jnp.max(p2, axis=1, keepdims=True)
    j2 = jnp.min(jnp.where(p2 == m2, colid, NE), axis=1, keepdims=True)
    sel2 = colid == j2
    w8 = jnp.where(sel1 | sel2, p, 0.0)

    maskf = w8[:, :NR] > 0
    maskv = maskf.astype(jnp.float32)

    r = lax.broadcasted_iota(jnp.int32, (TILE, TILE), 0)
    c = lax.broadcasted_iota(jnp.int32, (TILE, TILE), 1)
    tril = (c <= r).astype(jnp.float32)
    run = jnp.zeros((1, NR), jnp.float32)
    pos_tiles = []
    for i in range(SEQ // TILE):
        t = maskv[i * TILE:(i + 1) * TILE, :]
        pt = lax.dot_general(tril, t, (((1,), (0,)), ((), ())),
                             preferred_element_type=jnp.float32) + run
        run = run + jnp.sum(t, axis=0, keepdims=True)
        pos_tiles.append(pt)
    pos = jnp.concatenate(pos_tiles, axis=0)
    keep = maskf & (pos <= CAP)
    slot = pos - 1.0

    ecol = (lax.broadcasted_iota(jnp.int32, (SEQ, NR), 1) * CAP
            ).astype(jnp.float32)
    rowmat = jnp.where(keep, slot + ecol, jnp.inf)
    rv0 = jnp.min(rowmat, axis=1, keepdims=True)
    rv1 = jnp.min(jnp.where(rowmat == rv0, jnp.inf, rowmat),
                  axis=1, keepdims=True)
    dummy = jnp.mod(lax.broadcasted_iota(jnp.int32, (SEQ, 1), 0), GROWS
                    ).astype(jnp.float32)
    rm0_ref[...] = jnp.where(rv0 == jnp.inf, 0.0, 1.0)
    rm1_ref[...] = jnp.where(rv1 == jnp.inf, 0.0, 1.0)
    rv0_ref[...] = jnp.where(rv0 == jnp.inf, dummy, rv0).astype(jnp.int32)
    rv1_ref[...] = jnp.where(rv1 == jnp.inf, dummy, rv1).astype(jnp.int32)

    rowid = lax.broadcasted_iota(jnp.int32, (SEQ, 1), 0)
    rhi = (rowid // 256).astype(jnp.float32)
    rlo = (rowid % 256).astype(jnp.float32)
    ones = jnp.ones((SEQ, 1), jnp.float32)
    crange = lax.broadcasted_iota(jnp.int32, (SEQ, CAP), 1).astype(jnp.float32)
    srange = lax.broadcasted_iota(jnp.int32, (1, CAP), 1)
    for e in range(NR):
        oh = jnp.where((slot[:, e:e + 1] == crange) & keep[:, e:e + 1],
                       1.0, 0.0)
        we = w8[:, e:e + 1]
        whi = we.astype(jnp.bfloat16).astype(jnp.float32)
        rhs = jnp.concatenate([rhi, rlo, ones, whi, we - whi], axis=1)
        res = lax.dot_general(oh, rhs, (((0,), (0,)), ((), ())),
                              preferred_element_type=jnp.float32)
        tok = res[:, 0] * 256.0 + res[:, 1]
        dummy = (e * CAP + srange[0, :]) % SEQ
        gidx_ref[e, 0, :] = jnp.where(res[:, 2] > 0, tok.astype(jnp.int32),
                                      dummy)
        sw_ref[e, 0, :] = res[:, 3] + res[:, 4]
    gidx_ref[NR, 0, :] = srange[0, :]
    sw_ref[NR, 0, :] = jnp.zeros((CAP,), jnp.float32)
    noopw_ref[...] = w8[:, NR:NE]


def _ffn_body(xg_ref, sw_ref, w1_ref, w2_ref, y_ref):
    h = lax.dot_general(xg_ref[0].astype(jnp.bfloat16), w1_ref[0],
                        (((1,), (0,)), ((), ())),
                        preferred_element_type=jnp.float32)
    h = jnp.maximum(h, 0.0)
    y = lax.dot_general(h.astype(jnp.bfloat16), w2_ref[0],
                        (((1,), (0,)), ((), ())),
                        preferred_element_type=jnp.float32)
    y_ref[0] = y * sw_ref[0, 0, :][:, None]


def _add_body(x_ref, noopw_ref, rm0_ref, rm1_ref, yg0_ref, yg1_ref, out_ref):
    out_ref[...] = (x_ref[...] * noopw_ref[...]
                    + yg0_ref[...] * rm0_ref[...]
                    + yg1_ref[...] * rm1_ref[...])


_SC_MESH = plsc.VectorSubcoreMesh(core_axis_name="c", subcore_axis_name="s")


@functools.partial(
    pl.kernel, mesh=_SC_MESH,
    out_type=jax.ShapeDtypeStruct((GROWS, HID), jnp.float32),
    scratch_types=[
        pltpu.VMEM((GROWS // NW,), jnp.int32),
        pltpu.VMEM((GROWS // NW, HID), jnp.float32),
        pltpu.SemaphoreType.DMA,
    ],
)
def _sc_gather_x(xpad_hbm, gidx_hbm, xg_hbm, idx_v, rows_v, sem):
    wid = lax.axis_index("s") * NC + lax.axis_index("c")
    n = GROWS // NW
    base = wid * n
    pltpu.sync_copy(gidx_hbm.at[pl.ds(base, n)], idx_v)
    pltpu.async_copy(xpad_hbm.at[idx_v], rows_v, sem).wait()
    pltpu.sync_copy(rows_v, xg_hbm.at[pl.ds(base, n)])


@functools.partial(
    pl.kernel, mesh=_SC_MESH,
    out_type=jax.ShapeDtypeStruct((2 * SEQ, HID), jnp.float32),
    scratch_types=[
        pltpu.VMEM((TPW,), jnp.int32),
        pltpu.VMEM((TPW, HID), jnp.float32),
        pltpu.SemaphoreType.DMA,
    ],
)
def _sc_gather_y(ypad_hbm, rv0_hbm, rv1_hbm, yg_hbm, idx_v, rows_v, sem):
    wid = lax.axis_index("s") * NC + lax.axis_index("c")
    base = wid * TPW
    pltpu.sync_copy(rv0_hbm.at[pl.ds(base, TPW)], idx_v)
    pltpu.async_copy(ypad_hbm.at[idx_v], rows_v, sem).wait()
    pltpu.sync_copy(rows_v, yg_hbm.at[pl.ds(base, TPW)])
    pltpu.sync_copy(rv1_hbm.at[pl.ds(base, TPW)], idx_v)
    pltpu.async_copy(ypad_hbm.at[idx_v], rows_v, sem).wait()
    pltpu.sync_copy(rows_v, yg_hbm.at[pl.ds(SEQ + base, TPW)])


def kernel(x, W_router, b_router, experts_inter, experts_out):
    B, S, H = x.shape
    xf = x.reshape(S, H)

    gidx, sw, noopw, rv0, rv1, rm0, rm1 = pl.pallas_call(
        _router_body,
        out_shape=(
            jax.ShapeDtypeStruct((NE, 1, CAP), jnp.int32),
            jax.ShapeDtypeStruct((NE, 1, CAP), jnp.float32),
            jax.ShapeDtypeStruct((SEQ, 1), jnp.float32),
            jax.ShapeDtypeStruct((SEQ, 1), jnp.int32),
            jax.ShapeDtypeStruct((SEQ, 1), jnp.int32),
            jax.ShapeDtypeStruct((SEQ, 1), jnp.float32),
            jax.ShapeDtypeStruct((SEQ, 1), jnp.float32),
        ),
    )(xf, W_router, b_router.reshape(1, NE))

    gflat = gidx[:NR].reshape(GROWS)

    xg = _sc_gather_x(xf, gflat)

    y = pl.pallas_call(
        _ffn_body,
        grid=(NR,),
        in_specs=[
            pl.BlockSpec((1, CAP, HID), lambda e: (e, 0, 0)),
            pl.BlockSpec((1, 1, CAP), lambda e: (e, 0, 0)),
            pl.BlockSpec((1, HID, INTER), lambda e: (e, 0, 0)),
            pl.BlockSpec((1, INTER, HID), lambda e: (e, 0, 0)),
        ],
        out_specs=pl.BlockSpec((1, CAP, HID), lambda e: (e, 0, 0)),
        out_shape=jax.ShapeDtypeStruct((NR, CAP, HID), jnp.float32),
        compiler_params=pltpu.CompilerParams(
            dimension_semantics=("parallel",)),
    )(xg.reshape(NR, CAP, HID), sw,
      experts_inter.astype(jnp.bfloat16), experts_out.astype(jnp.bfloat16))

    yg = _sc_gather_y(y.reshape(GROWS, HID),
                      rv0.reshape(SEQ), rv1.reshape(SEQ))

    RB = 256
    out = pl.pallas_call(
        _add_body,
        grid=(SEQ // RB,),
        in_specs=[
            pl.BlockSpec((RB, HID), lambda i: (i, 0)),
            pl.BlockSpec((RB, 1), lambda i: (i, 0)),
            pl.BlockSpec((RB, 1), lambda i: (i, 0)),
            pl.BlockSpec((RB, 1), lambda i: (i, 0)),
            pl.BlockSpec((RB, HID), lambda i: (i, 0)),
            pl.BlockSpec((RB, HID), lambda i: (i + SEQ // RB, 0)),
        ],
        out_specs=pl.BlockSpec((RB, HID), lambda i: (i, 0)),
        out_shape=jax.ShapeDtypeStruct((SEQ, HID), jnp.float32),
        compiler_params=pltpu.CompilerParams(
            dimension_semantics=("parallel",)),
    )(xf, noopw, rm0, rm1, yg, yg)

    return out.reshape(B, S, H)


# speedup vs baseline: 1.5716x; 1.5716x over previous
import functools
import jax
import jax.numpy as jnp
from jax import lax
from jax.experimental import pallas as pl
from jax.experimental.pallas import tpu as pltpu
from jax.experimental.pallas import tpu_sc as plsc

NE = 8
NR = 7
CAP = 256
SEQ = 2048
HID = 1024
INTER = 2048
TILE = 256

NC = 2
NS = 16
NW = NC * NS
GROWS = NR * CAP


def _router_body(x_ref, wr_ref, br_ref, gidx_ref, sw_ref, xw_ref):
    x = x_ref[...]
    logits = lax.dot_general(
        x, wr_ref[...], (((1,), (1,)), ((), ())),
        preferred_element_type=jnp.float32) + br_ref[...]
    m = jnp.max(logits, axis=1, keepdims=True)
    ex = jnp.exp(logits - m)
    p = ex / jnp.sum(ex, axis=1, keepdims=True)

    colid = lax.broadcasted_iota(jnp.int32, (SEQ, NE), 1)
    m1 = jnp.max(p, axis=1, keepdims=True)
    j1 = jnp.min(jnp.where(p == m1, colid, NE), axis=1, keepdims=True)
    sel1 = colid == j1
    p2 = jnp.where(sel1, -jnp.inf, p)
    m2 = jnp.max(p2, axis=1, keepdims=True)
    j2 = jnp.min(jnp.where(p2 == m2, colid, NE), axis=1, keepdims=True)
    sel2 = colid == j2
    w8 = jnp.where(sel1 | sel2, p, 0.0)

    maskf = w8[:, :NR] > 0
    maskv = maskf.astype(jnp.float32)

    r = lax.broadcasted_iota(jnp.int32, (TILE, TILE), 0)
    c = lax.broadcasted_iota(jnp.int32, (TILE, TILE), 1)
    tril = (c <= r).astype(jnp.float32)
    run = jnp.zeros((1, NR), jnp.float32)
    pos_tiles = []
    for i in range(SEQ // TILE):
        t = maskv[i * TILE:(i + 1) * TILE, :]
        pt = lax.dot_general(tril, t, (((1,), (0,)), ((), ())),
                             preferred_element_type=jnp.float32) + run
        run = run + jnp.sum(t, axis=0, keepdims=True)
        pos_tiles.append(pt)
    pos = jnp.concatenate(pos_tiles, axis=0)
    keep = maskf & (pos <= CAP)
    slot = pos - 1.0

    rowid = lax.broadcasted_iota(jnp.int32, (SEQ, 1), 0)
    rhi = (rowid // 256).astype(jnp.float32)
    rlo = (rowid % 256).astype(jnp.float32)
    ones = jnp.ones((SEQ, 1), jnp.float32)
    crange = lax.broadcasted_iota(jnp.int32, (SEQ, CAP), 1).astype(jnp.float32)
    srange = lax.broadcasted_iota(jnp.int32, (1, CAP), 1)
    for e in range(NR):
        oh = jnp.where((slot[:, e:e + 1] == crange) & keep[:, e:e + 1],
                       1.0, 0.0)
        we = w8[:, e:e + 1]
        whi = we.astype(jnp.bfloat16).astype(jnp.float32)
        rhs = jnp.concatenate([rhi, rlo, ones, whi, we - whi], axis=1)
        res = lax.dot_general(oh, rhs, (((0,), (0,)), ((), ())),
                              preferred_element_type=jnp.float32)
        tok = res[:, 0] * 256.0 + res[:, 1]
        dummy = (e * CAP + srange[0, :]) % SEQ
        gidx_ref[e, 0, :] = jnp.where(res[:, 2] > 0, tok.astype(jnp.int32),
                                      dummy)
        sw_ref[e, 0, :] = res[:, 3] + res[:, 4]
    gidx_ref[NR, 0, :] = srange[0, :]
    sw_ref[NR, 0, :] = jnp.zeros((CAP,), jnp.float32)
    xw_ref[...] = x * w8[:, NR:NE]


def _ffn_body(xg_ref, gidx_ref, sw_ref, xw_ref,
              w1_ref, w2_ref, out_ref):
    e = pl.program_id(0)
    h = lax.dot_general(xg_ref[0], w1_ref[0], (((1,), (0,)), ((), ())),
                        preferred_element_type=jnp.float32)
    h = jnp.maximum(h, 0.0)
    y = lax.dot_general(h, w2_ref[0], (((1,), (0,)), ((), ())),
                        preferred_element_type=jnp.float32)
    y = y * sw_ref[0, 0, :][:, None]
    d = (lax.broadcasted_iota(jnp.int32, (CAP, SEQ), 1)
         == gidx_ref[0, 0, :][:, None]).astype(jnp.float32)
    contrib = lax.dot_general(d, y, (((0,), (0,)), ((), ())),
                              preferred_element_type=jnp.float32)

    @pl.when(e == 0)
    def _():
        out_ref[...] = xw_ref[...] + contrib

    @pl.when(e > 0)
    def _():
        out_ref[...] = out_ref[...] + contrib


_SC_MESH = plsc.VectorSubcoreMesh(core_axis_name="c", subcore_axis_name="s")


@functools.partial(
    pl.kernel, mesh=_SC_MESH,
    out_type=jax.ShapeDtypeStruct((GROWS, HID), jnp.float32),
    scratch_types=[
        pltpu.VMEM((GROWS // NW,), jnp.int32),
        pltpu.VMEM((GROWS // NW, HID), jnp.float32),
        pltpu.SemaphoreType.DMA,
    ],
)
def _sc_gather_x(xpad_hbm, gidx_hbm, xg_hbm, idx_v, rows_v, sem):
    wid = lax.axis_index("s") * NC + lax.axis_index("c")
    n = GROWS // NW
    base = wid * n
    pltpu.sync_copy(gidx_hbm.at[pl.ds(base, n)], idx_v)
    pltpu.async_copy(xpad_hbm.at[idx_v], rows_v, sem).wait()
    pltpu.sync_copy(rows_v, xg_hbm.at[pl.ds(base, n)])


def kernel(x, W_router, b_router, experts_inter, experts_out):
    B, S, H = x.shape
    xf = x.reshape(S, H)

    gidx, sw, xw = pl.pallas_call(
        _router_body,
        out_shape=(
            jax.ShapeDtypeStruct((NE, 1, CAP), jnp.int32),
            jax.ShapeDtypeStruct((NE, 1, CAP), jnp.float32),
            jax.ShapeDtypeStruct((SEQ, HID), jnp.float32),
        ),
    )(xf, W_router, b_router.reshape(1, NE))

    gflat = gidx[:NR].reshape(GROWS)

    xg = _sc_gather_x(xf, gflat)

    out = pl.pallas_call(
        _ffn_body,
        grid=(NR,),
        in_specs=[
            pl.BlockSpec((1, CAP, HID), lambda e: (e, 0, 0)),
            pl.BlockSpec((1, 1, CAP), lambda e: (e, 0, 0)),
            pl.BlockSpec((1, 1, CAP), lambda e: (e, 0, 0)),
            pl.BlockSpec((SEQ, HID), lambda e: (0, 0)),
            pl.BlockSpec((1, HID, INTER), lambda e: (e, 0, 0)),
            pl.BlockSpec((1, INTER, HID), lambda e: (e, 0, 0)),
        ],
        out_specs=pl.BlockSpec((SEQ, HID), lambda e: (0, 0)),
        out_shape=jax.ShapeDtypeStruct((SEQ, HID), jnp.float32),
        compiler_params=pltpu.CompilerParams(
            dimension_semantics=("arbitrary",)),
    )(xg.reshape(NR, CAP, HID), gidx, sw, xw,
      experts_inter, experts_out)

    return out.reshape(B, S, H)
